# baseline (speedup 1.0000x reference)
import functools

import jax
import jax.numpy as jnp
from jax import lax
from jax.experimental import pallas as pl
from jax.experimental.pallas import tpu as pltpu
from jax.experimental.pallas import tpu_sc as plsc

NUM_EMB = 8192
DIM = 256
BK = 2048
KB = NUM_EMB // BK
HW = 1024


BN = 128
GN = NUM_EMB // 128


def _dist_argmin_body(x_ref, cb2_ref, x2_ref, w2_ref, idx_ref, loss_ref):
    r = pl.program_id(0)

    xt = x_ref[0]
    cross2 = lax.dot_general(xt, cb2_ref[...], (((0,), (1,)), ((), ())),
                             preferred_element_type=jnp.float32)
    x2 = x2_ref[0]
    w2 = w2_ref[...]

    big = jnp.int32(2**31 - 1)
    iota_lane = lax.broadcasted_iota(jnp.int32, (1, 128), 1)
    lmin_parts = []
    lidx_parts = []
    for rb in range(BN // 8):
        rsl = slice(rb * 8, (rb + 1) * 8)
        x2b = x2[rsl, :]
        runmin = None
        runidx = None
        for g in range(GN):
            sl = slice(g * 128, (g + 1) * 128)
            d2 = (x2b + w2[:, sl]) - cross2[rsl, sl]
            dist = jnp.sqrt(jnp.maximum(d2, 0.0))
            ig = iota_lane + g * 128
            if g == 0:
                runmin = dist
                runidx = jnp.broadcast_to(ig, (8, 128))
            else:
                upd = dist < runmin
                runmin = jnp.where(upd, dist, runmin)
                runidx = jnp.where(upd, ig, runidx)
        lmin = jnp.min(runmin, axis=1, keepdims=True)
        lidx = jnp.min(jnp.where(runmin == lmin, runidx, big),
                       axis=1, keepdims=True)
        lmin_parts.append(lmin)
        lidx_parts.append(lidx)

    lmin_all = jnp.concatenate(lmin_parts, axis=0)
    idx_ref[0] = jnp.concatenate(lidx_parts, axis=0)

    s = jnp.sum(lmin_all * lmin_all, keepdims=True)

    @pl.when(r == 0)
    def _():
        loss_ref[...] = s

    @pl.when(r > 0)
    def _():
        loss_ref[...] = loss_ref[...] + s


def _dist_argmin(xr, cb2, x2, w2):
    B = xr.shape[0]
    grid = (B * HW // BN,)
    nsub = HW // BN
    out = pl.pallas_call(
        _dist_argmin_body,
        grid=grid,
        in_specs=[
            pl.BlockSpec((1, DIM, BN), lambda r: (r // nsub, 0, r % nsub)),
            pl.BlockSpec((NUM_EMB, DIM), lambda r: (0, 0)),
            pl.BlockSpec((1, BN, 1), lambda r: (r // nsub, r % nsub, 0)),
            pl.BlockSpec((1, NUM_EMB), lambda r: (0, 0)),
        ],
        out_specs=[
            pl.BlockSpec((1, BN, 1), lambda r: (r // nsub, r % nsub, 0)),
            pl.BlockSpec((1, 1), lambda r: (0, 0)),
        ],
        out_shape=[
            jax.ShapeDtypeStruct((B, HW, 1), jnp.int32),
            jax.ShapeDtypeStruct((1, 1), jnp.float32),
        ],
    )(xr, cb2, x2, w2)
    return out


_SC_WORKERS = 32
_BPW = (8 * HW) // _SC_WORKERS


@functools.lru_cache(maxsize=1)
def _make_sc_gather():
    @functools.partial(
        pl.kernel,
        mesh=plsc.VectorSubcoreMesh(core_axis_name="c", subcore_axis_name="s"),
        out_type=jax.ShapeDtypeStruct((8 * HW, DIM), jnp.float32),
        scratch_types=[
            pltpu.VMEM((_BPW,), jnp.int32),
            pltpu.VMEM((_BPW, DIM), jnp.float32),
            pltpu.SemaphoreType.DMA,
        ],
    )
    def _sc_gather(table_hbm, idx_hbm, out_hbm, idx_v, rows_v, sem):
        wid = lax.axis_index("s") * 2 + lax.axis_index("c")
        base = wid * _BPW
        pltpu.sync_copy(idx_hbm.at[pl.ds(base, _BPW)], idx_v)
        pltpu.async_copy(table_hbm.at[idx_v], rows_v, sem).wait()
        pltpu.sync_copy(rows_v, out_hbm.at[pl.ds(base, _BPW)])

    return _sc_gather


def kernel(x, codebook):
    B, C, H, W = x.shape
    hw = H * W
    xr = x.reshape(B, C, hw)
    xf = jnp.transpose(xr, (0, 2, 1))
    x2 = jnp.sum(xf ** 2, axis=-1, keepdims=True)
    w2 = jnp.sum(codebook ** 2, axis=-1).reshape(1, NUM_EMB)
    cb2 = codebook * 2.0

    idx, loss_sum = _dist_argmin(xr, cb2, x2, w2)

    idx_flat = idx.reshape(B * hw)
    quant = _make_sc_gather()(codebook, idx_flat)

    quant_out = jnp.transpose(quant.reshape(B, hw, C), (0, 2, 1)).reshape(
        B, C, H, W)
    loss = loss_sum[0, 0] / jnp.float32(B * hw * C)
    indices = idx.reshape(B, H, W)
    return quant_out, loss, loss, indices


# speedup vs baseline: 1.1452x; 1.1452x over previous
import functools

import jax
import jax.numpy as jnp
from jax import lax
from jax.experimental import pallas as pl
from jax.experimental.pallas import tpu as pltpu
from jax.experimental.pallas import tpu_sc as plsc

NUM_EMB = 8192
DIM = 256
BK = 2048
KB = NUM_EMB // BK
HW = 1024


BN = 256
GN = NUM_EMB // 128


def _dist_argmin_body(x_ref, cb2_ref, x2_ref, w2_ref, idx_ref, loss_ref):
    r = pl.program_id(0)

    xt = x_ref[0]
    cross2 = lax.dot_general(xt, cb2_ref[...], (((0,), (1,)), ((), ())),
                             preferred_element_type=jnp.float32)
    x2 = x2_ref[0]
    w2 = w2_ref[...]

    big = jnp.int32(2**31 - 1)
    iota_lane = lax.broadcasted_iota(jnp.int32, (1, 128), 1)
    lmin_parts = []
    lidx_parts = []
    for rb in range(BN // 8):
        rsl = slice(rb * 8, (rb + 1) * 8)
        x2b = x2[rsl, :]
        runmin = None
        runidx = None
        for g in range(GN):
            sl = slice(g * 128, (g + 1) * 128)
            d2 = (x2b + w2[:, sl]) - cross2[rsl, sl]
            dist = jnp.sqrt(jnp.maximum(d2, 0.0))
            ig = iota_lane + g * 128
            if g == 0:
                runmin = dist
                runidx = jnp.broadcast_to(ig, (8, 128))
            else:
                upd = dist < runmin
                runmin = jnp.where(upd, dist, runmin)
                runidx = jnp.where(upd, ig, runidx)
        lmin = jnp.min(runmin, axis=1, keepdims=True)
        lidx = jnp.min(jnp.where(runmin == lmin, runidx, big),
                       axis=1, keepdims=True)
        lmin_parts.append(lmin)
        lidx_parts.append(lidx)

    lmin_all = jnp.concatenate(lmin_parts, axis=0)
    idx_ref[0] = jnp.concatenate(lidx_parts, axis=0)

    s = jnp.sum(lmin_all * lmin_all, keepdims=True)

    @pl.when(r == 0)
    def _():
        loss_ref[...] = s

    @pl.when(r > 0)
    def _():
        loss_ref[...] = loss_ref[...] + s


def _dist_argmin(xr, cb2, x2, w2):
    B = xr.shape[0]
    grid = (B * HW // BN,)
    nsub = HW // BN
    out = pl.pallas_call(
        _dist_argmin_body,
        grid=grid,
        in_specs=[
            pl.BlockSpec((1, DIM, BN), lambda r: (r // nsub, 0, r % nsub)),
            pl.BlockSpec((NUM_EMB, DIM), lambda r: (0, 0)),
            pl.BlockSpec((1, BN, 1), lambda r: (r // nsub, r % nsub, 0)),
            pl.BlockSpec((1, NUM_EMB), lambda r: (0, 0)),
        ],
        out_specs=[
            pl.BlockSpec((1, BN, 1), lambda r: (r // nsub, r % nsub, 0)),
            pl.BlockSpec((1, 1), lambda r: (0, 0)),
        ],
        out_shape=[
            jax.ShapeDtypeStruct((B, HW, 1), jnp.int32),
            jax.ShapeDtypeStruct((1, 1), jnp.float32),
        ],
    )(xr, cb2, x2, w2)
    return out


_SC_WORKERS = 32
_BPW = (8 * HW) // _SC_WORKERS


@functools.lru_cache(maxsize=1)
def _make_sc_gather():
    @functools.partial(
        pl.kernel,
        mesh=plsc.VectorSubcoreMesh(core_axis_name="c", subcore_axis_name="s"),
        out_type=jax.ShapeDtypeStruct((8 * HW, DIM), jnp.float32),
        scratch_types=[
            pltpu.VMEM((_BPW,), jnp.int32),
            pltpu.VMEM((_BPW, DIM), jnp.float32),
            pltpu.SemaphoreType.DMA,
        ],
    )
    def _sc_gather(table_hbm, idx_hbm, out_hbm, idx_v, rows_v, sem):
        wid = lax.axis_index("s") * 2 + lax.axis_index("c")
        base = wid * _BPW
        pltpu.sync_copy(idx_hbm.at[pl.ds(base, _BPW)], idx_v)
        pltpu.async_copy(table_hbm.at[idx_v], rows_v, sem).wait()
        pltpu.sync_copy(rows_v, out_hbm.at[pl.ds(base, _BPW)])

    return _sc_gather


def kernel(x, codebook):
    B, C, H, W = x.shape
    hw = H * W
    xr = x.reshape(B, C, hw)
    xf = jnp.transpose(xr, (0, 2, 1))
    x2 = jnp.sum(xf ** 2, axis=-1, keepdims=True)
    w2 = jnp.sum(codebook ** 2, axis=-1).reshape(1, NUM_EMB)
    cb2 = codebook * 2.0

    idx, loss_sum = _dist_argmin(xr, cb2, x2, w2)

    idx_flat = idx.reshape(B * hw)
    quant = _make_sc_gather()(codebook, idx_flat)

    quant_out = jnp.transpose(quant.reshape(B, hw, C), (0, 2, 1)).reshape(
        B, C, H, W)
    loss = loss_sum[0, 0] / jnp.float32(B * hw * C)
    indices = idx.reshape(B, H, W)
    return quant_out, loss, loss, indices


# speedup vs baseline: 1.2057x; 1.0529x over previous
import functools

import jax
import jax.numpy as jnp
from jax import lax
from jax.experimental import pallas as pl
from jax.experimental.pallas import tpu as pltpu
from jax.experimental.pallas import tpu_sc as plsc

NUM_EMB = 8192
DIM = 256
BK = 2048
KB = NUM_EMB // BK
HW = 1024


BN = 512
GN = NUM_EMB // 128


def _dist_argmin_body(x_ref, cb2_ref, x2_ref, w2_ref, idx_ref, loss_ref):
    r = pl.program_id(0)

    xt = x_ref[0]
    cross2 = lax.dot_general(xt, cb2_ref[...], (((0,), (1,)), ((), ())),
                             preferred_element_type=jnp.float32)
    x2 = x2_ref[0]
    w2 = w2_ref[...]

    big = jnp.int32(2**31 - 1)
    iota_lane = lax.broadcasted_iota(jnp.int32, (1, 128), 1)
    lmin_parts = []
    lidx_parts = []
    for rb in range(BN // 8):
        rsl = slice(rb * 8, (rb + 1) * 8)
        x2b = x2[rsl, :]
        runmin = None
        runidx = None
        for g in range(GN):
            sl = slice(g * 128, (g + 1) * 128)
            d2 = (x2b + w2[:, sl]) - cross2[rsl, sl]
            dist = jnp.sqrt(jnp.maximum(d2, 0.0))
            ig = iota_lane + g * 128
            if g == 0:
                runmin = dist
                runidx = jnp.broadcast_to(ig, (8, 128))
            else:
                upd = dist < runmin
                runmin = jnp.where(upd, dist, runmin)
                runidx = jnp.where(upd, ig, runidx)
        lmin = jnp.min(runmin, axis=1, keepdims=True)
        lidx = jnp.min(jnp.where(runmin == lmin, runidx, big),
                       axis=1, keepdims=True)
        lmin_parts.append(lmin)
        lidx_parts.append(lidx)

    lmin_all = jnp.concatenate(lmin_parts, axis=0)
    idx_ref[0] = jnp.concatenate(lidx_parts, axis=0)

    s = jnp.sum(lmin_all * lmin_all, keepdims=True)

    @pl.when(r == 0)
    def _():
        loss_ref[...] = s

    @pl.when(r > 0)
    def _():
        loss_ref[...] = loss_ref[...] + s


def _dist_argmin(xr, cb2, x2, w2):
    B = xr.shape[0]
    grid = (B * HW // BN,)
    nsub = HW // BN
    out = pl.pallas_call(
        _dist_argmin_body,
        grid=grid,
        in_specs=[
            pl.BlockSpec((1, DIM, BN), lambda r: (r // nsub, 0, r % nsub)),
            pl.BlockSpec((NUM_EMB, DIM), lambda r: (0, 0)),
            pl.BlockSpec((1, BN, 1), lambda r: (r // nsub, r % nsub, 0)),
            pl.BlockSpec((1, NUM_EMB), lambda r: (0, 0)),
        ],
        out_specs=[
            pl.BlockSpec((1, BN, 1), lambda r: (r // nsub, r % nsub, 0)),
            pl.BlockSpec((1, 1), lambda r: (0, 0)),
        ],
        out_shape=[
            jax.ShapeDtypeStruct((B, HW, 1), jnp.int32),
            jax.ShapeDtypeStruct((1, 1), jnp.float32),
        ],
    )(xr, cb2, x2, w2)
    return out


_SC_WORKERS = 32
_BPW = (8 * HW) // _SC_WORKERS


@functools.lru_cache(maxsize=1)
def _make_sc_gather():
    @functools.partial(
        pl.kernel,
        mesh=plsc.VectorSubcoreMesh(core_axis_name="c", subcore_axis_name="s"),
        out_type=jax.ShapeDtypeStruct((8 * HW, DIM), jnp.float32),
        scratch_types=[
            pltpu.VMEM((_BPW,), jnp.int32),
            pltpu.VMEM((_BPW, DIM), jnp.float32),
            pltpu.SemaphoreType.DMA,
        ],
    )
    def _sc_gather(table_hbm, idx_hbm, out_hbm, idx_v, rows_v, sem):
        wid = lax.axis_index("s") * 2 + lax.axis_index("c")
        base = wid * _BPW
        pltpu.sync_copy(idx_hbm.at[pl.ds(base, _BPW)], idx_v)
        pltpu.async_copy(table_hbm.at[idx_v], rows_v, sem).wait()
        pltpu.sync_copy(rows_v, out_hbm.at[pl.ds(base, _BPW)])

    return _sc_gather


def kernel(x, codebook):
    B, C, H, W = x.shape
    hw = H * W
    xr = x.reshape(B, C, hw)
    xf = jnp.transpose(xr, (0, 2, 1))
    x2 = jnp.sum(xf ** 2, axis=-1, keepdims=True)
    w2 = jnp.sum(codebook ** 2, axis=-1).reshape(1, NUM_EMB)
    cb2 = codebook * 2.0

    idx, loss_sum = _dist_argmin(xr, cb2, x2, w2)

    idx_flat = idx.reshape(B * hw)
    quant = _make_sc_gather()(codebook, idx_flat)

    quant_out = jnp.transpose(quant.reshape(B, hw, C), (0, 2, 1)).reshape(
        B, C, H, W)
    loss = loss_sum[0, 0] / jnp.float32(B * hw * C)
    indices = idx.reshape(B, H, W)
    return quant_out, loss, loss, indices


# speedup vs baseline: 1.3010x; 1.0790x over previous
import functools

import jax
import jax.numpy as jnp
from jax import lax
from jax.experimental import pallas as pl
from jax.experimental.pallas import tpu as pltpu
from jax.experimental.pallas import tpu_sc as plsc

NUM_EMB = 8192
DIM = 256
BK = 2048
KB = NUM_EMB // BK
HW = 1024


BN = 1024
GN = NUM_EMB // 128


def _dist_argmin_body(x_ref, cb2_ref, x2_ref, w2_ref, idx_ref, loss_ref):
    r = pl.program_id(0)

    xt = x_ref[0]
    cross2 = lax.dot_general(xt, cb2_ref[...], (((0,), (1,)), ((), ())),
                             preferred_element_type=jnp.float32)
    x2 = x2_ref[0]
    w2 = w2_ref[...]

    big = jnp.int32(2**31 - 1)
    iota_lane = lax.broadcasted_iota(jnp.int32, (1, 128), 1)
    lmin_parts = []
    lidx_parts = []
    for rb in range(BN // 8):
        rsl = slice(rb * 8, (rb + 1) * 8)
        x2b = x2[rsl, :]
        runmin = None
        runidx = None
        for g in range(GN):
            sl = slice(g * 128, (g + 1) * 128)
            d2 = (x2b + w2[:, sl]) - cross2[rsl, sl]
            dist = jnp.sqrt(jnp.maximum(d2, 0.0))
            ig = iota_lane + g * 128
            if g == 0:
                runmin = dist
                runidx = jnp.broadcast_to(ig, (8, 128))
            else:
                upd = dist < runmin
                runmin = jnp.where(upd, dist, runmin)
                runidx = jnp.where(upd, ig, runidx)
        lmin = jnp.min(runmin, axis=1, keepdims=True)
        lidx = jnp.min(jnp.where(runmin == lmin, runidx, big),
                       axis=1, keepdims=True)
        lmin_parts.append(lmin)
        lidx_parts.append(lidx)

    lmin_all = jnp.concatenate(lmin_parts, axis=0)
    idx_ref[0] = jnp.concatenate(lidx_parts, axis=0)

    s = jnp.sum(lmin_all * lmin_all, keepdims=True)

    @pl.when(r == 0)
    def _():
        loss_ref[...] = s

    @pl.when(r > 0)
    def _():
        loss_ref[...] = loss_ref[...] + s


def _dist_argmin(xr, cb2, x2, w2):
    B = xr.shape[0]
    grid = (B * HW // BN,)
    nsub = HW // BN
    out = pl.pallas_call(
        _dist_argmin_body,
        grid=grid,
        in_specs=[
            pl.BlockSpec((1, DIM, BN), lambda r: (r // nsub, 0, r % nsub)),
            pl.BlockSpec((NUM_EMB, DIM), lambda r: (0, 0)),
            pl.BlockSpec((1, BN, 1), lambda r: (r // nsub, r % nsub, 0)),
            pl.BlockSpec((1, NUM_EMB), lambda r: (0, 0)),
        ],
        out_specs=[
            pl.BlockSpec((1, BN, 1), lambda r: (r // nsub, r % nsub, 0)),
            pl.BlockSpec((1, 1), lambda r: (0, 0)),
        ],
        out_shape=[
            jax.ShapeDtypeStruct((B, HW, 1), jnp.int32),
            jax.ShapeDtypeStruct((1, 1), jnp.float32),
        ],
    )(xr, cb2, x2, w2)
    return out


_SC_WORKERS = 32
_BPW = (8 * HW) // _SC_WORKERS


@functools.lru_cache(maxsize=1)
def _make_sc_gather():
    @functools.partial(
        pl.kernel,
        mesh=plsc.VectorSubcoreMesh(core_axis_name="c", subcore_axis_name="s"),
        out_type=jax.ShapeDtypeStruct((8 * HW, DIM), jnp.float32),
        scratch_types=[
            pltpu.VMEM((_BPW,), jnp.int32),
            pltpu.VMEM((_BPW, DIM), jnp.float32),
            pltpu.SemaphoreType.DMA,
        ],
    )
    def _sc_gather(table_hbm, idx_hbm, out_hbm, idx_v, rows_v, sem):
        wid = lax.axis_index("s") * 2 + lax.axis_index("c")
        base = wid * _BPW
        pltpu.sync_copy(idx_hbm.at[pl.ds(base, _BPW)], idx_v)
        pltpu.async_copy(table_hbm.at[idx_v], rows_v, sem).wait()
        pltpu.sync_copy(rows_v, out_hbm.at[pl.ds(base, _BPW)])

    return _sc_gather


def kernel(x, codebook):
    B, C, H, W = x.shape
    hw = H * W
    xr = x.reshape(B, C, hw)
    xf = jnp.transpose(xr, (0, 2, 1))
    x2 = jnp.sum(xf ** 2, axis=-1, keepdims=True)
    w2 = jnp.sum(codebook ** 2, axis=-1).reshape(1, NUM_EMB)
    cb2 = codebook * 2.0

    idx, loss_sum = _dist_argmin(xr, cb2, x2, w2)

    idx_flat = idx.reshape(B * hw)
    quant = _make_sc_gather()(codebook, idx_flat)

    quant_out = jnp.transpose(quant.reshape(B, hw, C), (0, 2, 1)).reshape(
        B, C, H, W)
    loss = loss_sum[0, 0] / jnp.float32(B * hw * C)
    indices = idx.reshape(B, H, W)
    return quant_out, loss, loss, indices


# speedup vs baseline: 1.7302x; 1.3299x over previous
import functools

import jax
import jax.numpy as jnp
from jax import lax
from jax.experimental import pallas as pl
from jax.experimental.pallas import tpu as pltpu
from jax.experimental.pallas import tpu_sc as plsc

NUM_EMB = 8192
DIM = 256
BK = 2048
KB = NUM_EMB // BK
HW = 1024


BN = 1024
GN = NUM_EMB // 128


def _dist_argmin_body(x_ref, cb2_ref, x2_ref, w2_ref, idx_ref, loss_ref,
                      flag_ref):
    r = pl.program_id(0)

    xt = x_ref[0]
    cross2 = lax.dot_general(xt, cb2_ref[...], (((0,), (1,)), ((), ())),
                             preferred_element_type=jnp.float32)
    x2 = x2_ref[0]
    w2 = w2_ref[...]

    big = jnp.int32(2**31 - 1)
    inf = jnp.float32(jnp.inf)
    iota_lane = lax.broadcasted_iota(jnp.int32, (1, 128), 1)

    lmin_parts = []
    lidx_parts = []
    flag_parts = []
    for rb in range(BN // 8):
        rsl = slice(rb * 8, (rb + 1) * 8)
        x2b = x2[rsl, :]
        m1 = None
        m2 = None
        ridx = None
        for g in range(GN):
            sl = slice(g * 128, (g + 1) * 128)
            d2 = (x2b + w2[:, sl]) - cross2[rsl, sl]
            ig = iota_lane + g * 128
            if g == 0:
                m1 = d2
                m2 = jnp.full((8, 128), inf, jnp.float32)
                ridx = jnp.broadcast_to(ig, (8, 128))
            else:
                upd = d2 < m1
                m2 = jnp.minimum(m2, jnp.maximum(m1, d2))
                m1 = jnp.minimum(m1, d2)
                ridx = jnp.where(upd, ig, ridx)
        gm1 = jnp.min(m1, axis=1, keepdims=True)
        eq1 = m1 == gm1
        gidx = jnp.min(jnp.where(eq1, ridx, big),
                       axis=1, keepdims=True)
        nties = jnp.sum(eq1.astype(jnp.float32), axis=1, keepdims=True)
        rest = jnp.min(jnp.where(eq1, inf, m1), axis=1, keepdims=True)
        gm2 = jnp.minimum(jnp.min(m2, axis=1, keepdims=True), rest)
        gm2 = jnp.where(nties > 1.0, gm1, gm2)
        s1 = jnp.sqrt(jnp.maximum(gm1, 0.0))
        s2 = jnp.sqrt(jnp.maximum(gm2, 0.0))
        flag = (s1 == s2) | (gm1 <= 0.0)
        lmin_parts.append(s1)
        lidx_parts.append(gidx)
        flag_parts.append(flag)

    lmin_all = jnp.concatenate(lmin_parts, axis=0)
    idx_ref[0] = jnp.concatenate(lidx_parts, axis=0)
    flags = jnp.concatenate(flag_parts, axis=0)
    f = jnp.sum(jnp.where(flags, 1.0, 0.0), keepdims=True)
    s = jnp.sum(lmin_all * lmin_all, keepdims=True)

    @pl.when(r == 0)
    def _():
        loss_ref[...] = s
        flag_ref[...] = f

    @pl.when(r > 0)
    def _():
        loss_ref[...] = loss_ref[...] + s
        flag_ref[...] = flag_ref[...] + f


EBN = 512


def _exact_body(x_ref, cb2_ref, x2_ref, w2_ref, idx_ref, loss_ref):
    r = pl.program_id(0)

    xt = x_ref[0]
    cross2 = lax.dot_general(xt, cb2_ref[...], (((0,), (1,)), ((), ())),
                             preferred_element_type=jnp.float32)
    x2 = x2_ref[0]
    w2 = w2_ref[...]

    big = jnp.int32(2**31 - 1)
    iota_lane = lax.broadcasted_iota(jnp.int32, (1, 128), 1)
    lmin_parts = []
    lidx_parts = []
    for rb in range(EBN // 8):
        rsl = slice(rb * 8, (rb + 1) * 8)
        x2b = x2[rsl, :]
        runmin = None
        runidx = None
        for g in range(GN):
            sl = slice(g * 128, (g + 1) * 128)
            d2 = (x2b + w2[:, sl]) - cross2[rsl, sl]
            dist = jnp.sqrt(jnp.maximum(d2, 0.0))
            ig = iota_lane + g * 128
            if g == 0:
                runmin = dist
                runidx = jnp.broadcast_to(ig, (8, 128))
            else:
                upd = dist < runmin
                runmin = jnp.where(upd, dist, runmin)
                runidx = jnp.where(upd, ig, runidx)
        lmin = jnp.min(runmin, axis=1, keepdims=True)
        lidx = jnp.min(jnp.where(runmin == lmin, runidx, big),
                       axis=1, keepdims=True)
        lmin_parts.append(lmin)
        lidx_parts.append(lidx)

    lmin_all = jnp.concatenate(lmin_parts, axis=0)
    idx_ref[0] = jnp.concatenate(lidx_parts, axis=0)
    s = jnp.sum(lmin_all * lmin_all, keepdims=True)

    @pl.when(r == 0)
    def _():
        loss_ref[...] = s

    @pl.when(r > 0)
    def _():
        loss_ref[...] = loss_ref[...] + s


def _dist_argmin(xr, cb2, x2, w2):
    B = xr.shape[0]
    grid = (B * HW // BN,)
    nsub = HW // BN
    idx, loss_sum, flag = pl.pallas_call(
        _dist_argmin_body,
        grid=grid,
        in_specs=[
            pl.BlockSpec((1, DIM, BN), lambda r: (r // nsub, 0, r % nsub)),
            pl.BlockSpec((NUM_EMB, DIM), lambda r: (0, 0)),
            pl.BlockSpec((1, BN, 1), lambda r: (r // nsub, r % nsub, 0)),
            pl.BlockSpec((1, NUM_EMB), lambda r: (0, 0)),
        ],
        out_specs=[
            pl.BlockSpec((1, BN, 1), lambda r: (r // nsub, r % nsub, 0)),
            pl.BlockSpec((1, 1), lambda r: (0, 0)),
            pl.BlockSpec((1, 1), lambda r: (0, 0)),
        ],
        out_shape=[
            jax.ShapeDtypeStruct((B, HW, 1), jnp.int32),
            jax.ShapeDtypeStruct((1, 1), jnp.float32),
            jax.ShapeDtypeStruct((1, 1), jnp.float32),
        ],
    )(xr, cb2, x2, w2)

    def _exact(_):
        ensub = HW // EBN
        return pl.pallas_call(
            _exact_body,
            grid=(B * HW // EBN,),
            in_specs=[
                pl.BlockSpec((1, DIM, EBN),
                             lambda r: (r // ensub, 0, r % ensub)),
                pl.BlockSpec((NUM_EMB, DIM), lambda r: (0, 0)),
                pl.BlockSpec((1, EBN, 1),
                             lambda r: (r // ensub, r % ensub, 0)),
                pl.BlockSpec((1, NUM_EMB), lambda r: (0, 0)),
            ],
            out_specs=[
                pl.BlockSpec((1, EBN, 1),
                             lambda r: (r // ensub, r % ensub, 0)),
                pl.BlockSpec((1, 1), lambda r: (0, 0)),
            ],
            out_shape=[
                jax.ShapeDtypeStruct((B, HW, 1), jnp.int32),
                jax.ShapeDtypeStruct((1, 1), jnp.float32),
            ],
        )(xr, cb2, x2, w2)

    idx, loss_sum = lax.cond(flag[0, 0] > 0.0, _exact,
                             lambda _: (idx, loss_sum), None)
    return idx, loss_sum


_SC_WORKERS = 32
_BPW = (8 * HW) // _SC_WORKERS


@functools.lru_cache(maxsize=1)
def _make_sc_gather():
    @functools.partial(
        pl.kernel,
        mesh=plsc.VectorSubcoreMesh(core_axis_name="c", subcore_axis_name="s"),
        out_type=jax.ShapeDtypeStruct((8 * HW, DIM), jnp.float32),
        scratch_types=[
            pltpu.VMEM((_BPW,), jnp.int32),
            pltpu.VMEM((_BPW, DIM), jnp.float32),
            pltpu.SemaphoreType.DMA,
        ],
    )
    def _sc_gather(table_hbm, idx_hbm, out_hbm, idx_v, rows_v, sem):
        wid = lax.axis_index("s") * 2 + lax.axis_index("c")
        base = wid * _BPW
        pltpu.sync_copy(idx_hbm.at[pl.ds(base, _BPW)], idx_v)
        pltpu.async_copy(table_hbm.at[idx_v], rows_v, sem).wait()
        pltpu.sync_copy(rows_v, out_hbm.at[pl.ds(base, _BPW)])

    return _sc_gather


def kernel(x, codebook):
    B, C, H, W = x.shape
    hw = H * W
    xr = x.reshape(B, C, hw)
    xf = jnp.transpose(xr, (0, 2, 1))
    x2 = jnp.sum(xf ** 2, axis=-1, keepdims=True)
    w2 = jnp.sum(codebook ** 2, axis=-1).reshape(1, NUM_EMB)
    cb2 = codebook * 2.0

    idx, loss_sum = _dist_argmin(xr, cb2, x2, w2)

    idx_flat = idx.reshape(B * hw)
    quant = _make_sc_gather()(codebook, idx_flat)

    quant_out = jnp.transpose(quant.reshape(B, hw, C), (0, 2, 1)).reshape(
        B, C, H, W)
    loss = loss_sum[0, 0] / jnp.float32(B * hw * C)
    indices = idx.reshape(B, H, W)
    return quant_out, loss, loss, indices


# speedup vs baseline: 1.7601x; 1.0173x over previous
import functools

import jax
import jax.numpy as jnp
from jax import lax
from jax.experimental import pallas as pl
from jax.experimental.pallas import tpu as pltpu
from jax.experimental.pallas import tpu_sc as plsc

NUM_EMB = 8192
DIM = 256
BK = 2048
KB = NUM_EMB // BK
HW = 1024


BN = 1024
GN = NUM_EMB // 128


def _dist_argmin_body(x_ref, cb_ref, x2h_ref, w2h_ref, idx_ref, loss_ref,
                      flag_ref):
    r = pl.program_id(0)

    xt = x_ref[0]
    cross = lax.dot_general(xt, cb_ref[...], (((0,), (1,)), ((), ())),
                            preferred_element_type=jnp.float32)
    x2 = x2h_ref[0]
    w2 = w2h_ref[...]

    big = jnp.int32(2**31 - 1)
    inf = jnp.float32(jnp.inf)
    iota_lane = lax.broadcasted_iota(jnp.int32, (1, 128), 1)

    lmin_parts = []
    lidx_parts = []
    flag_parts = []
    for rb in range(BN // 8):
        rsl = slice(rb * 8, (rb + 1) * 8)
        x2b = x2[rsl, :]
        m1 = None
        m2 = None
        ridx = None
        for g in range(GN):
            sl = slice(g * 128, (g + 1) * 128)
            d2 = (x2b + w2[:, sl]) - cross[rsl, sl]
            ig = iota_lane + g * 128
            if g == 0:
                m1 = d2
                m2 = jnp.full((8, 128), inf, jnp.float32)
                ridx = jnp.broadcast_to(ig, (8, 128))
            else:
                upd = d2 < m1
                m2 = jnp.minimum(m2, jnp.maximum(m1, d2))
                m1 = jnp.minimum(m1, d2)
                ridx = jnp.where(upd, ig, ridx)
        gm1 = jnp.min(m1, axis=1, keepdims=True)
        eq1 = m1 == gm1
        gidx = jnp.min(jnp.where(eq1, ridx, big),
                       axis=1, keepdims=True)
        nties = jnp.sum(eq1.astype(jnp.float32), axis=1, keepdims=True)
        rest = jnp.min(jnp.where(eq1, inf, m1), axis=1, keepdims=True)
        gm2 = jnp.minimum(jnp.min(m2, axis=1, keepdims=True), rest)
        gm2 = jnp.where(nties > 1.0, gm1, gm2)
        s1 = jnp.sqrt(jnp.maximum(2.0 * gm1, 0.0))
        s2 = jnp.sqrt(jnp.maximum(2.0 * gm2, 0.0))
        flag = (s1 == s2) | (gm1 <= 0.0)
        lmin_parts.append(s1)
        lidx_parts.append(gidx)
        flag_parts.append(flag)

    lmin_all = jnp.concatenate(lmin_parts, axis=0)
    idx_ref[0] = jnp.concatenate(lidx_parts, axis=0)
    flags = jnp.concatenate(flag_parts, axis=0)
    f = jnp.sum(jnp.where(flags, 1.0, 0.0), keepdims=True)
    s = jnp.sum(lmin_all * lmin_all, keepdims=True)

    @pl.when(r == 0)
    def _():
        loss_ref[...] = s
        flag_ref[...] = f

    @pl.when(r > 0)
    def _():
        loss_ref[...] = loss_ref[...] + s
        flag_ref[...] = flag_ref[...] + f


EBN = 512


def _exact_body(x_ref, cb_ref, x2h_ref, w2h_ref, idx_ref, loss_ref):
    r = pl.program_id(0)

    xt = x_ref[0]
    cross = lax.dot_general(xt, cb_ref[...], (((0,), (1,)), ((), ())),
                            preferred_element_type=jnp.float32)
    x2 = x2h_ref[0]
    w2 = w2h_ref[...]

    big = jnp.int32(2**31 - 1)
    iota_lane = lax.broadcasted_iota(jnp.int32, (1, 128), 1)
    lmin_parts = []
    lidx_parts = []
    for rb in range(EBN // 8):
        rsl = slice(rb * 8, (rb + 1) * 8)
        x2b = x2[rsl, :]
        runmin = None
        runidx = None
        for g in range(GN):
            sl = slice(g * 128, (g + 1) * 128)
            d2h = (x2b + w2[:, sl]) - cross[rsl, sl]
            dist = jnp.sqrt(jnp.maximum(2.0 * d2h, 0.0))
            ig = iota_lane + g * 128
            if g == 0:
                runmin = dist
                runidx = jnp.broadcast_to(ig, (8, 128))
            else:
                upd = dist < runmin
                runmin = jnp.where(upd, dist, runmin)
                runidx = jnp.where(upd, ig, runidx)
        lmin = jnp.min(runmin, axis=1, keepdims=True)
        lidx = jnp.min(jnp.where(runmin == lmin, runidx, big),
                       axis=1, keepdims=True)
        lmin_parts.append(lmin)
        lidx_parts.append(lidx)

    lmin_all = jnp.concatenate(lmin_parts, axis=0)
    idx_ref[0] = jnp.concatenate(lidx_parts, axis=0)
    s = jnp.sum(lmin_all * lmin_all, keepdims=True)

    @pl.when(r == 0)
    def _():
        loss_ref[...] = s

    @pl.when(r > 0)
    def _():
        loss_ref[...] = loss_ref[...] + s


def _dist_argmin(xr, codebook, x2h, w2h):
    B = xr.shape[0]
    grid = (B * HW // BN,)
    nsub = HW // BN
    idx, loss_sum, flag = pl.pallas_call(
        _dist_argmin_body,
        grid=grid,
        in_specs=[
            pl.BlockSpec((1, DIM, BN), lambda r: (r // nsub, 0, r % nsub)),
            pl.BlockSpec((NUM_EMB, DIM), lambda r: (0, 0)),
            pl.BlockSpec((1, BN, 1), lambda r: (r // nsub, r % nsub, 0)),
            pl.BlockSpec((1, NUM_EMB), lambda r: (0, 0)),
        ],
        out_specs=[
            pl.BlockSpec((1, BN, 1), lambda r: (r // nsub, r % nsub, 0)),
            pl.BlockSpec((1, 1), lambda r: (0, 0)),
            pl.BlockSpec((1, 1), lambda r: (0, 0)),
        ],
        out_shape=[
            jax.ShapeDtypeStruct((B, HW, 1), jnp.int32),
            jax.ShapeDtypeStruct((1, 1), jnp.float32),
            jax.ShapeDtypeStruct((1, 1), jnp.float32),
        ],
    )(xr, codebook, x2h, w2h)

    def _exact(_):
        ensub = HW // EBN
        return pl.pallas_call(
            _exact_body,
            grid=(B * HW // EBN,),
            in_specs=[
                pl.BlockSpec((1, DIM, EBN),
                             lambda r: (r // ensub, 0, r % ensub)),
                pl.BlockSpec((NUM_EMB, DIM), lambda r: (0, 0)),
                pl.BlockSpec((1, EBN, 1),
                             lambda r: (r // ensub, r % ensub, 0)),
                pl.BlockSpec((1, NUM_EMB), lambda r: (0, 0)),
            ],
            out_specs=[
                pl.BlockSpec((1, EBN, 1),
                             lambda r: (r // ensub, r % ensub, 0)),
                pl.BlockSpec((1, 1), lambda r: (0, 0)),
            ],
            out_shape=[
                jax.ShapeDtypeStruct((B, HW, 1), jnp.int32),
                jax.ShapeDtypeStruct((1, 1), jnp.float32),
            ],
        )(xr, codebook, x2h, w2h)

    idx, loss_sum = lax.cond(flag[0, 0] > 0.0, _exact,
                             lambda _: (idx, loss_sum), None)
    return idx, loss_sum


_SC_WORKERS = 32
_BPW = (8 * HW) // _SC_WORKERS


@functools.lru_cache(maxsize=1)
def _make_sc_gather():
    @functools.partial(
        pl.kernel,
        mesh=plsc.VectorSubcoreMesh(core_axis_name="c", subcore_axis_name="s"),
        out_type=jax.ShapeDtypeStruct((8 * HW, DIM), jnp.float32),
        scratch_types=[
            pltpu.VMEM((_BPW,), jnp.int32),
            pltpu.VMEM((_BPW, DIM), jnp.float32),
            pltpu.SemaphoreType.DMA,
        ],
    )
    def _sc_gather(table_hbm, idx_hbm, out_hbm, idx_v, rows_v, sem):
        wid = lax.axis_index("s") * 2 + lax.axis_index("c")
        base = wid * _BPW
        pltpu.sync_copy(idx_hbm.at[pl.ds(base, _BPW)], idx_v)
        pltpu.async_copy(table_hbm.at[idx_v], rows_v, sem).wait()
        pltpu.sync_copy(rows_v, out_hbm.at[pl.ds(base, _BPW)])

    return _sc_gather


def kernel(x, codebook):
    B, C, H, W = x.shape
    hw = H * W
    xr = x.reshape(B, C, hw)
    xf = jnp.transpose(xr, (0, 2, 1))
    x2 = jnp.sum(xf ** 2, axis=-1, keepdims=True)
    w2 = jnp.sum(codebook ** 2, axis=-1).reshape(1, NUM_EMB)

    idx, loss_sum = _dist_argmin(xr, codebook, x2 * 0.5, w2 * 0.5)

    idx_flat = idx.reshape(B * hw)
    quant = _make_sc_gather()(codebook, idx_flat)

    quant_out = jnp.transpose(quant.reshape(B, hw, C), (0, 2, 1)).reshape(
        B, C, H, W)
    loss = loss_sum[0, 0] / jnp.float32(B * hw * C)
    indices = idx.reshape(B, H, W)
    return quant_out, loss, loss, indices


# speedup vs baseline: 1.7816x; 1.0122x over previous
import functools

import jax
import jax.numpy as jnp
from jax import lax
from jax.experimental import pallas as pl
from jax.experimental.pallas import tpu as pltpu
from jax.experimental.pallas import tpu_sc as plsc

NUM_EMB = 8192
DIM = 256
BK = 2048
KB = NUM_EMB // BK
HW = 1024


BN = 1024
GN = NUM_EMB // 128


def _dist_argmin_body(x_ref, cb_ref, x2h_ref, w2h_ref, idx_ref, loss_ref,
                      flag_ref):
    r = pl.program_id(0)

    xt = x_ref[0]
    cross = lax.dot_general(xt, cb_ref[...], (((0,), (1,)), ((), ())),
                            preferred_element_type=jnp.float32)
    x2 = x2h_ref[0]
    w2 = w2h_ref[...]

    big = jnp.int32(2**31 - 1)
    inf = jnp.float32(jnp.inf)
    iota_lane = lax.broadcasted_iota(jnp.int32, (1, 128), 1)

    lmin_parts = []
    lidx_parts = []
    flag_parts = []
    for rb in range(BN // 8):
        rsl = slice(rb * 8, (rb + 1) * 8)
        x2b = x2[rsl, :]
        m1 = None
        m2 = None
        ridx = None
        for g in range(GN):
            sl = slice(g * 128, (g + 1) * 128)
            d2 = (x2b + w2[:, sl]) - cross[rsl, sl]
            ig = iota_lane + g * 128
            if g == 0:
                m1 = d2
                m2 = jnp.full((8, 128), inf, jnp.float32)
                ridx = jnp.broadcast_to(ig, (8, 128))
            else:
                upd = d2 < m1
                m2 = jnp.minimum(m2, jnp.maximum(m1, d2))
                m1 = jnp.minimum(m1, d2)
                ridx = jnp.where(upd, ig, ridx)
        gm1 = jnp.min(m1, axis=1, keepdims=True)
        eq1 = m1 == gm1
        gidx = jnp.min(jnp.where(eq1, ridx, big),
                       axis=1, keepdims=True)
        nties = jnp.sum(eq1.astype(jnp.float32), axis=1, keepdims=True)
        rest = jnp.min(jnp.where(eq1, inf, m1), axis=1, keepdims=True)
        gm2 = jnp.minimum(jnp.min(m2, axis=1, keepdims=True), rest)
        gm2 = jnp.where(nties > 1.0, gm1, gm2)
        s1 = jnp.sqrt(jnp.maximum(2.0 * gm1, 0.0))
        s2 = jnp.sqrt(jnp.maximum(2.0 * gm2, 0.0))
        flag = (s1 == s2) | (gm1 <= 0.0)
        lmin_parts.append(s1)
        lidx_parts.append(gidx)
        flag_parts.append(flag)

    lmin_all = jnp.concatenate(lmin_parts, axis=0)
    idx_ref[0] = jnp.concatenate(lidx_parts, axis=0)
    flags = jnp.concatenate(flag_parts, axis=0)
    f = jnp.sum(jnp.where(flags, 1.0, 0.0), keepdims=True)
    s = jnp.sum(lmin_all * lmin_all, keepdims=True)

    @pl.when(r == 0)
    def _():
        loss_ref[...] = s
        flag_ref[...] = f

    @pl.when(r > 0)
    def _():
        loss_ref[...] = loss_ref[...] + s
        flag_ref[...] = flag_ref[...] + f


EBN = 512


def _exact_body(x_ref, cb_ref, x2h_ref, w2h_ref, idx_ref, loss_ref):
    r = pl.program_id(0)

    xt = x_ref[0]
    cross = lax.dot_general(xt, cb_ref[...], (((0,), (1,)), ((), ())),
                            preferred_element_type=jnp.float32)
    x2 = x2h_ref[0]
    w2 = w2h_ref[...]

    big = jnp.int32(2**31 - 1)
    iota_lane = lax.broadcasted_iota(jnp.int32, (1, 128), 1)
    lmin_parts = []
    lidx_parts = []
    for rb in range(EBN // 8):
        rsl = slice(rb * 8, (rb + 1) * 8)
        x2b = x2[rsl, :]
        runmin = None
        runidx = None
        for g in range(GN):
            sl = slice(g * 128, (g + 1) * 128)
            d2h = (x2b + w2[:, sl]) - cross[rsl, sl]
            dist = jnp.sqrt(jnp.maximum(2.0 * d2h, 0.0))
            ig = iota_lane + g * 128
            if g == 0:
                runmin = dist
                runidx = jnp.broadcast_to(ig, (8, 128))
            else:
                upd = dist < runmin
                runmin = jnp.where(upd, dist, runmin)
                runidx = jnp.where(upd, ig, runidx)
        lmin = jnp.min(runmin, axis=1, keepdims=True)
        lidx = jnp.min(jnp.where(runmin == lmin, runidx, big),
                       axis=1, keepdims=True)
        lmin_parts.append(lmin)
        lidx_parts.append(lidx)

    lmin_all = jnp.concatenate(lmin_parts, axis=0)
    idx_ref[0] = jnp.concatenate(lidx_parts, axis=0)
    s = jnp.sum(lmin_all * lmin_all, keepdims=True)

    @pl.when(r == 0)
    def _():
        loss_ref[...] = s

    @pl.when(r > 0)
    def _():
        loss_ref[...] = loss_ref[...] + s


def _dist_argmin(xr, codebook, x2h, w2h):
    B = xr.shape[0]
    grid = (B * HW // BN,)
    nsub = HW // BN
    idx, loss_sum, flag = pl.pallas_call(
        _dist_argmin_body,
        grid=grid,
        in_specs=[
            pl.BlockSpec((1, DIM, BN), lambda r: (r // nsub, 0, r % nsub)),
            pl.BlockSpec((NUM_EMB, DIM), lambda r: (0, 0)),
            pl.BlockSpec((1, BN, 1), lambda r: (r // nsub, r % nsub, 0)),
            pl.BlockSpec((1, NUM_EMB), lambda r: (0, 0)),
        ],
        out_specs=[
            pl.BlockSpec((1, BN, 1), lambda r: (r // nsub, r % nsub, 0)),
            pl.BlockSpec((1, 1), lambda r: (0, 0)),
            pl.BlockSpec((1, 1), lambda r: (0, 0)),
        ],
        out_shape=[
            jax.ShapeDtypeStruct((B, HW, 1), jnp.int32),
            jax.ShapeDtypeStruct((1, 1), jnp.float32),
            jax.ShapeDtypeStruct((1, 1), jnp.float32),
        ],
    )(xr, codebook, x2h, w2h)

    def _exact(_):
        ensub = HW // EBN
        return pl.pallas_call(
            _exact_body,
            grid=(B * HW // EBN,),
            in_specs=[
                pl.BlockSpec((1, DIM, EBN),
                             lambda r: (r // ensub, 0, r % ensub)),
                pl.BlockSpec((NUM_EMB, DIM), lambda r: (0, 0)),
                pl.BlockSpec((1, EBN, 1),
                             lambda r: (r // ensub, r % ensub, 0)),
                pl.BlockSpec((1, NUM_EMB), lambda r: (0, 0)),
            ],
            out_specs=[
                pl.BlockSpec((1, EBN, 1),
                             lambda r: (r // ensub, r % ensub, 0)),
                pl.BlockSpec((1, 1), lambda r: (0, 0)),
            ],
            out_shape=[
                jax.ShapeDtypeStruct((B, HW, 1), jnp.int32),
                jax.ShapeDtypeStruct((1, 1), jnp.float32),
            ],
        )(xr, codebook, x2h, w2h)

    idx, loss_sum = lax.cond(flag[0, 0] > 0.0, _exact,
                             lambda _: (idx, loss_sum), None)
    return idx, loss_sum


_SC_WORKERS = 32
_BPW = (8 * HW) // _SC_WORKERS


@functools.lru_cache(maxsize=1)
def _make_sc_gather():
    @functools.partial(
        pl.kernel,
        mesh=plsc.VectorSubcoreMesh(core_axis_name="c", subcore_axis_name="s"),
        out_type=jax.ShapeDtypeStruct((8 * HW, DIM), jnp.float32),
        scratch_types=[
            pltpu.VMEM((_BPW,), jnp.int32),
            pltpu.VMEM((_BPW, DIM), jnp.float32),
            pltpu.SemaphoreType.DMA,
        ],
    )
    def _sc_gather(table_hbm, idx_hbm, out_hbm, idx_v, rows_v, sem):
        wid = lax.axis_index("s") * 2 + lax.axis_index("c")
        base = wid * _BPW
        pltpu.sync_copy(idx_hbm.at[pl.ds(base, _BPW)], idx_v)
        pltpu.async_copy(table_hbm.at[idx_v], rows_v, sem).wait()
        pltpu.sync_copy(rows_v, out_hbm.at[pl.ds(base, _BPW)])

    return _sc_gather


def kernel(x, codebook):
    B, C, H, W = x.shape
    hw = H * W
    xr = x.reshape(B, C, hw)
    x2 = jnp.sum(xr ** 2, axis=1)[..., None]
    w2 = jnp.sum(codebook ** 2, axis=-1).reshape(1, NUM_EMB)

    idx, loss_sum = _dist_argmin(xr, codebook, x2 * 0.5, w2 * 0.5)

    idx_flat = idx.reshape(B * hw)
    quant = _make_sc_gather()(codebook, idx_flat)

    quant_out = jnp.transpose(quant.reshape(B, hw, C), (0, 2, 1)).reshape(
        B, C, H, W)
    loss = loss_sum[0, 0] / jnp.float32(B * hw * C)
    indices = idx.reshape(B, H, W)
    return quant_out, loss, loss, indices
